# Initial kernel scaffold; baseline (speedup 1.0000x reference)
import jax
import jax.numpy as jnp
from jax.experimental import pallas as pl


def kernel(indices, table, W, b):
    raise NotImplementedError("write your pallas kernel here")


# speedup vs baseline: 1.1964x; 1.1964x over previous
import functools

import jax
import jax.numpy as jnp
from jax import lax
from jax.experimental import pallas as pl
from jax.experimental.pallas import tpu as pltpu
from jax.experimental.pallas import tpu_sc as plsc

CH = 128
G = 4
ROWS_PER_GROUP = CH * G


@functools.cache
def _make_sc_gather(NW, NGRP, D):
    mesh = plsc.VectorSubcoreMesh(core_axis_name="c", subcore_axis_name="s")
    info = plsc.get_sparse_core_info()
    NC = info.num_cores

    @functools.partial(
        pl.kernel,
        mesh=mesh,
        compiler_params=pltpu.CompilerParams(use_tc_tiling_on_sc=False),
        out_type=jax.ShapeDtypeStruct((NW, NGRP, ROWS_PER_GROUP, D), jnp.float32),
        scratch_types=[
            pltpu.VMEM((NGRP, G, CH), jnp.int32),
            pltpu.VMEM((2, ROWS_PER_GROUP, D), jnp.float32),
            pltpu.SemaphoreType.DMA,
            pltpu.SemaphoreType.DMA,
        ],
    )
    def sc_gather(table_hbm, idx_hbm, out_hbm, idx_v, rows_v, sem0, sem1):
        wid = lax.axis_index("s") * NC + lax.axis_index("c")
        pltpu.sync_copy(idx_hbm.at[wid], idx_v)
        sems = (sem0, sem1)

        def fire(g, b):
            for j in range(G):
                pltpu.async_copy(
                    table_hbm.at[idx_v.at[g, j]],
                    rows_v.at[b, pl.ds(j * CH, CH)],
                    sems[b],
                )

        def drain(b):
            pltpu.make_async_copy(
                table_hbm.at[pl.ds(0, ROWS_PER_GROUP)], rows_v.at[b], sems[b]
            ).wait()

        fire(0, 0)
        fire(1, 1)

        def body(i, carry):
            for b in range(2):
                g = 2 * i + b
                drain(b)
                pltpu.sync_copy(rows_v.at[b], out_hbm.at[wid, g])
                fire(g + 2, b)
            return carry

        lax.fori_loop(0, NGRP // 2 - 1, body, 0)
        for b in range(2):
            g = NGRP - 2 + b
            drain(b)
            pltpu.sync_copy(rows_v.at[b], out_hbm.at[wid, g])

    return sc_gather


def _mm_body(x_ref, w_ref, b_ref, o_ref):
    o_ref[...] = (
        jnp.dot(x_ref[...], w_ref[...], preferred_element_type=jnp.float32)
        + b_ref[...]
    )


def _project(x_flat, W, b, blk):
    n = x_flat.shape[0]
    d_in = W.shape[0]
    d_out = W.shape[1]
    return pl.pallas_call(
        _mm_body,
        grid=(n // blk,),
        in_specs=[
            pl.BlockSpec((blk, d_in), lambda i: (i, 0)),
            pl.BlockSpec((d_in, d_out), lambda i: (0, 0)),
            pl.BlockSpec((1, d_out), lambda i: (0, 0)),
        ],
        out_specs=pl.BlockSpec((blk, d_out), lambda i: (i, 0)),
        out_shape=jax.ShapeDtypeStruct((n, d_out), jnp.float32),
    )(x_flat, W, b.reshape(1, d_out))


def kernel(indices, table, W, b):
    batch, seq = indices.shape
    vocab, d = table.shape
    n_rows = batch * seq

    info = plsc.get_sparse_core_info()
    NW = info.num_cores * info.num_subcores
    per_w = n_rows // NW
    assert per_w * NW == n_rows and per_w % ROWS_PER_GROUP == 0
    ngrp = per_w // ROWS_PER_GROUP

    idx4 = indices.astype(jnp.int32).reshape(NW, ngrp, G, CH)
    gathered = _make_sc_gather(NW, ngrp, d)(table, idx4)
    x_flat = gathered.reshape(n_rows, d)
    out = _project(x_flat, W, b, blk=8192)
    return out.reshape(batch, seq, W.shape[1])
